# Initial kernel scaffold; baseline (speedup 1.0000x reference)
import functools

import jax
import jax.numpy as jnp
import numpy as np
from jax import lax
from jax.experimental import pallas as pl
from jax.experimental.pallas import tpu as pltpu
from jax.experimental.pallas import tpu_sc as plsc

B = 16384
F = 26
V = 100000
D = 16
H = 300
FD = F * D

NC = 2
NS = 16
NW = NC * NS
RPW = (B * F) // NW
IDX_MINOR = 128
IDX_ROWS = RPW // IDX_MINOR
GATHERS_PER_CHUNK = 13
NCHUNK = IDX_ROWS // GATHERS_PER_CHUNK
CROWS = GATHERS_PER_CHUNK * IDX_MINOR


def _sc_gather_body(idx_hbm, off_hbm, tab_hbm, out_hbm, idx_raw, idx_flat,
                    off_v, rows_v, sem):
    w = lax.axis_index("s") * NC + lax.axis_index("c")
    pltpu.sync_copy(idx_hbm.at[w], idx_raw)
    pltpu.sync_copy(off_hbm, off_v)

    def add_row(r, carry):
        for g in range(IDX_MINOR // 16):
            sl = pl.ds(g * 16, 16)
            idx_flat[r, sl] = idx_raw[r, sl] + off_v[r, sl]
        return carry

    lax.fori_loop(0, IDX_ROWS, add_row, 0)

    def do_chunk(c, carry):
        cps = []
        for j in range(GATHERS_PER_CHUNK):
            cp = pltpu.make_async_copy(
                tab_hbm.at[idx_flat.at[c * GATHERS_PER_CHUNK + j]],
                rows_v.at[pl.ds(j * IDX_MINOR, IDX_MINOR)],
                sem,
            )
            cp.start()
            cps.append(cp)
        for cp in cps:
            cp.wait()
        pltpu.sync_copy(rows_v, out_hbm.at[pl.ds(w * RPW + c * CROWS, CROWS)])
        return carry

    lax.fori_loop(0, NCHUNK, do_chunk, 0)


_sc_gather = pl.kernel(
    _sc_gather_body,
    out_type=jax.ShapeDtypeStruct((B * F, D), jnp.float32),
    mesh=plsc.VectorSubcoreMesh(core_axis_name="c", subcore_axis_name="s"),
    scratch_types=[
        pltpu.VMEM((IDX_ROWS, IDX_MINOR), jnp.int32),
        pltpu.VMEM((IDX_ROWS, IDX_MINOR), jnp.int32),
        pltpu.VMEM((IDX_ROWS, IDX_MINOR), jnp.int32),
        pltpu.VMEM((CROWS, D), jnp.float32),
        pltpu.SemaphoreType.DMA,
    ],
)

_OFFS = np.tile(np.arange(F, dtype=np.int32) * V,
                RPW // F).reshape(IDX_ROWS, IDX_MINOR)

TB = 2048
NSTEP = B // TB


def _stats_body(emb_ref, w1_ref, b1_ref, gamma_ref, beta_ref, w2_ref, b2_ref,
                wvec_ref, k_ref, sum_ref, sumsq_ref):
    i = pl.program_id(0)

    @pl.when(i == 0)
    def _():
        sum_ref[...] = jnp.zeros_like(sum_ref)
        sumsq_ref[...] = jnp.zeros_like(sumsq_ref)

    blk = emb_ref[...]
    h0 = lax.dot_general(blk, w1_ref[...], (((1,), (0,)), ((), ())),
                         preferred_element_type=jnp.float32)
    sum_ref[...] += jnp.sum(h0, axis=0, keepdims=True)
    sumsq_ref[...] += jnp.sum(h0 * h0, axis=0, keepdims=True)

    @pl.when(i == NSTEP - 1)
    def _():
        b1 = b1_ref[...]
        w2 = w2_ref[...]
        s0 = sum_ref[...]
        mean0 = s0 * (1.0 / B)
        mean = mean0 + b1
        var = sumsq_ref[...] * (1.0 / B) - mean0 * mean0
        c = gamma_ref[...] * w2 * lax.rsqrt(var + 1e-5)
        wvec_ref[...] = lax.dot_general(c, w1_ref[...], (((1,), (1,)), ((), ())),
                                        preferred_element_type=jnp.float32)
        k_ref[...] = b2_ref[...] + jnp.sum(
            beta_ref[...] * w2 + (b1 - mean) * c, axis=1, keepdims=True)


_stats_call = pl.pallas_call(
    _stats_body,
    grid=(NSTEP,),
    in_specs=[
        pl.BlockSpec((TB, FD), lambda i: (i, 0)),
        pl.BlockSpec((FD, H), lambda i: (0, 0)),
        pl.BlockSpec((1, H), lambda i: (0, 0)),
        pl.BlockSpec((1, H), lambda i: (0, 0)),
        pl.BlockSpec((1, H), lambda i: (0, 0)),
        pl.BlockSpec((1, H), lambda i: (0, 0)),
        pl.BlockSpec((1, 1), lambda i: (0, 0)),
    ],
    out_specs=[
        pl.BlockSpec((1, FD), lambda i: (0, 0)),
        pl.BlockSpec((1, 1), lambda i: (0, 0)),
    ],
    out_shape=[
        jax.ShapeDtypeStruct((1, FD), jnp.float32),
        jax.ShapeDtypeStruct((1, 1), jnp.float32),
    ],
    scratch_shapes=[
        pltpu.VMEM((1, H), jnp.float32),
        pltpu.VMEM((1, H), jnp.float32),
    ],
    compiler_params=pltpu.CompilerParams(
        dimension_semantics=("arbitrary",)),
)


def _out_body(emb_ref, wvec_ref, k_ref, out_ref):
    z = lax.dot_general(emb_ref[...], wvec_ref[...], (((1,), (1,)), ((), ())),
                        preferred_element_type=jnp.float32)
    out_ref[...] = jax.nn.sigmoid(z + k_ref[...])


_out_call = pl.pallas_call(
    _out_body,
    grid=(NSTEP,),
    in_specs=[
        pl.BlockSpec((TB, FD), lambda i: (i, 0)),
        pl.BlockSpec((1, FD), lambda i: (0, 0)),
        pl.BlockSpec((1, 1), lambda i: (0, 0)),
    ],
    out_specs=pl.BlockSpec((TB, 1), lambda i: (i, 0)),
    out_shape=jax.ShapeDtypeStruct((B, 1), jnp.float32),
)


def kernel(inputs, tables, W1, b1, gamma, beta, W2, b2):
    idx3 = inputs.reshape(NW, IDX_ROWS, IDX_MINOR).astype(jnp.int32)
    offs = jnp.asarray(_OFFS)
    tab_flat = tables.reshape(F * V, D)
    emb_flat = _sc_gather(idx3, offs, tab_flat)
    emb = emb_flat.reshape(B, FD)
    wvec, k = _stats_call(emb, W1, b1.reshape(1, H), gamma.reshape(1, H),
                          gamma.reshape(1, H), W2.reshape(1, H),
                          b2.reshape(1, 1))
    return _out_call(emb, wvec, k)


# speedup vs baseline: 2.1459x; 2.1459x over previous
import functools

import jax
import jax.numpy as jnp
import numpy as np
from jax import lax
from jax.experimental import pallas as pl
from jax.experimental.pallas import tpu as pltpu
from jax.experimental.pallas import tpu_sc as plsc

B = 16384
F = 26
V = 100000
D = 16
H = 300
FD = F * D

NC = 2
NS = 16
NW = NC * NS
RPW = (B * F) // NW
IDX_MINOR = 128
IDX_ROWS = RPW // IDX_MINOR
GATHERS_PER_CHUNK = 13
NCHUNK = IDX_ROWS // GATHERS_PER_CHUNK
CROWS = GATHERS_PER_CHUNK * IDX_MINOR


def _sc_gather_body(idx_hbm, off_hbm, tab_hbm, out_hbm, idx_raw, idx_flat,
                    off_v, rows_v, sem):
    w = lax.axis_index("s") * NC + lax.axis_index("c")
    pltpu.sync_copy(idx_hbm.at[w], idx_raw)
    pltpu.sync_copy(off_hbm, off_v)

    def add_row(r, carry):
        for g in range(IDX_MINOR // 16):
            sl = pl.ds(g * 16, 16)
            idx_flat[r, sl] = idx_raw[r, sl] + off_v[r, sl]
        return carry

    lax.fori_loop(0, IDX_ROWS, add_row, 0)

    def do_chunk(c, carry):
        cps = []
        for j in range(GATHERS_PER_CHUNK):
            cp = pltpu.make_async_copy(
                tab_hbm.at[idx_flat.at[c * GATHERS_PER_CHUNK + j]],
                rows_v.at[pl.ds(j * IDX_MINOR, IDX_MINOR)],
                sem,
            )
            cp.start()
            cps.append(cp)
        for cp in cps:
            cp.wait()
        pltpu.sync_copy(rows_v, out_hbm.at[pl.ds(w * RPW + c * CROWS, CROWS)])
        return carry

    lax.fori_loop(0, NCHUNK, do_chunk, 0)


@functools.cache
def _get_sc_gather():
    return pl.kernel(
        _sc_gather_body,
        out_type=jax.ShapeDtypeStruct((B * F, D), jnp.float32),
        mesh=plsc.VectorSubcoreMesh(core_axis_name="c", subcore_axis_name="s"),
        scratch_types=[
            pltpu.VMEM((IDX_ROWS, IDX_MINOR), jnp.int32),
            pltpu.VMEM((IDX_ROWS, IDX_MINOR), jnp.int32),
            pltpu.VMEM((IDX_ROWS, IDX_MINOR), jnp.int32),
            pltpu.VMEM((CROWS, D), jnp.float32),
            pltpu.SemaphoreType.DMA,
        ],
        compiler_params=pltpu.CompilerParams(use_tc_tiling_on_sc=False),
    )

_OFFS = np.tile(np.arange(F, dtype=np.int32) * V,
                RPW // F).reshape(IDX_ROWS, IDX_MINOR)

TB = 2048
NSTEP = B // TB


def _stats_body(emb_ref, w1_ref, b1_ref, gamma_ref, beta_ref, w2_ref, b2_ref,
                wvec_ref, k_ref, sum_ref, sumsq_ref):
    i = pl.program_id(0)

    @pl.when(i == 0)
    def _():
        sum_ref[...] = jnp.zeros_like(sum_ref)
        sumsq_ref[...] = jnp.zeros_like(sumsq_ref)

    blk = emb_ref[...]
    h0 = lax.dot_general(blk, w1_ref[...], (((1,), (0,)), ((), ())),
                         preferred_element_type=jnp.float32)
    sum_ref[...] += jnp.sum(h0, axis=0, keepdims=True)
    sumsq_ref[...] += jnp.sum(h0 * h0, axis=0, keepdims=True)

    @pl.when(i == NSTEP - 1)
    def _():
        b1 = b1_ref[...]
        w2 = w2_ref[...]
        s0 = sum_ref[...]
        mean0 = s0 * (1.0 / B)
        mean = mean0 + b1
        var = sumsq_ref[...] * (1.0 / B) - mean0 * mean0
        c = gamma_ref[...] * w2 * lax.rsqrt(var + 1e-5)
        wvec_ref[...] = lax.dot_general(c, w1_ref[...], (((1,), (1,)), ((), ())),
                                        preferred_element_type=jnp.float32)
        k_ref[...] = b2_ref[...] + jnp.sum(
            beta_ref[...] * w2 + (b1 - mean) * c, axis=1, keepdims=True)


_stats_call = pl.pallas_call(
    _stats_body,
    grid=(NSTEP,),
    in_specs=[
        pl.BlockSpec((TB, FD), lambda i: (i, 0)),
        pl.BlockSpec((FD, H), lambda i: (0, 0)),
        pl.BlockSpec((1, H), lambda i: (0, 0)),
        pl.BlockSpec((1, H), lambda i: (0, 0)),
        pl.BlockSpec((1, H), lambda i: (0, 0)),
        pl.BlockSpec((1, H), lambda i: (0, 0)),
        pl.BlockSpec((1, 1), lambda i: (0, 0)),
    ],
    out_specs=[
        pl.BlockSpec((1, FD), lambda i: (0, 0)),
        pl.BlockSpec((1, 1), lambda i: (0, 0)),
    ],
    out_shape=[
        jax.ShapeDtypeStruct((1, FD), jnp.float32),
        jax.ShapeDtypeStruct((1, 1), jnp.float32),
    ],
    scratch_shapes=[
        pltpu.VMEM((1, H), jnp.float32),
        pltpu.VMEM((1, H), jnp.float32),
    ],
    compiler_params=pltpu.CompilerParams(
        dimension_semantics=("arbitrary",)),
)


def _out_body(emb_ref, wvec_ref, k_ref, out_ref):
    z = jnp.sum(emb_ref[...] * wvec_ref[...], axis=1, keepdims=True)
    out_ref[...] = jax.nn.sigmoid(z + k_ref[0, 0])


_out_call = pl.pallas_call(
    _out_body,
    grid=(NSTEP,),
    in_specs=[
        pl.BlockSpec((TB, FD), lambda i: (i, 0)),
        pl.BlockSpec((1, FD), lambda i: (0, 0)),
        pl.BlockSpec((1, 1), lambda i: (0, 0)),
    ],
    out_specs=pl.BlockSpec((TB, 1), lambda i: (i, 0)),
    out_shape=jax.ShapeDtypeStruct((B, 1), jnp.float32),
)


def kernel(inputs, tables, W1, b1, gamma, beta, W2, b2):
    idx3 = inputs.reshape(NW, IDX_ROWS, IDX_MINOR).astype(jnp.int32)
    offs = jnp.asarray(_OFFS)
    tab_flat = tables.reshape(F * V, D)
    emb_flat = _get_sc_gather()(idx3, offs, tab_flat)
    emb = emb_flat.reshape(B, FD)
    wvec, k = _stats_call(emb, W1, b1.reshape(1, H), gamma.reshape(1, H),
                          beta.reshape(1, H), W2.reshape(1, H),
                          b2.reshape(1, 1))
    return _out_call(emb, wvec, k)


# speedup vs baseline: 5.6871x; 2.6502x over previous
import functools

import jax
import jax.numpy as jnp
import numpy as np
from jax import lax
from jax.experimental import pallas as pl
from jax.experimental.pallas import tpu as pltpu
from jax.experimental.pallas import tpu_sc as plsc

B = 16384
F = 26
V = 100000
D = 16
H = 300
FD = F * D

NC = 2
NS = 16
NW = NC * NS
PPW = FD // NW
QV = 4096
NQ = B // QV


def _sc_gather_body(idx_hbm, tab_hbm, out_hbm, idx_v, plane_v, stage_v,
                    sem0, sem1):
    w = lax.axis_index("s") * NC + lax.axis_index("c")
    p0 = w * PPW
    sems = (sem0, sem1)
    pending = [None, None]

    for j in range(PPW):
        p = p0 + j
        f = p // D
        if j == 0:
            pltpu.sync_copy(idx_hbm.at[f], idx_v)
        else:
            f_prev = (p - 1) // D
            @pl.when(f != f_prev)
            def _():
                pltpu.sync_copy(idx_hbm.at[f], idx_v)
        pltpu.sync_copy(tab_hbm.at[p], plane_v)

        for q in range(NQ):
            s = (j * NQ + q) % 2
            if pending[s] is not None:
                pending[s].wait()

            def gq(i, carry):
                b = i * 32
                v0 = idx_v[pl.ds(q * QV + b, 16)]
                v1 = idx_v[pl.ds(q * QV + b + 16, 16)]
                stage_v[s, pl.ds(b, 16)] = plsc.load_gather(plane_v, [v0])
                stage_v[s, pl.ds(b + 16, 16)] = plsc.load_gather(plane_v, [v1])
                return carry

            lax.fori_loop(0, QV // 32, gq, 0)
            cp = pltpu.make_async_copy(
                stage_v.at[s], out_hbm.at[p, pl.ds(q * QV, QV)], sems[s])
            cp.start()
            pending[s] = cp

    for s in range(2):
        if pending[s] is not None:
            pending[s].wait()


@functools.cache
def _get_sc_gather():
    return pl.kernel(
        _sc_gather_body,
        out_type=jax.ShapeDtypeStruct((FD, B), jnp.float32),
        mesh=plsc.VectorSubcoreMesh(core_axis_name="c", subcore_axis_name="s"),
        scratch_types=[
            pltpu.VMEM((B,), jnp.int32),
            pltpu.VMEM((V,), jnp.float32),
            pltpu.VMEM((2, QV), jnp.float32),
            pltpu.SemaphoreType.DMA,
            pltpu.SemaphoreType.DMA,
        ],
        compiler_params=pltpu.CompilerParams(use_tc_tiling_on_sc=False,
                                             needs_layout_passes=False),
    )


TB = 2048
NSTEP = B // TB


def _stats_body(emb_ref, w1_ref, b1_ref, gamma_ref, beta_ref, w2_ref, b2_ref,
                wvec_ref, k_ref, sum_ref, sumsq_ref):
    i = pl.program_id(0)

    @pl.when(i == 0)
    def _():
        sum_ref[...] = jnp.zeros_like(sum_ref)
        sumsq_ref[...] = jnp.zeros_like(sumsq_ref)

    blk = emb_ref[...]
    h0 = lax.dot_general(blk, w1_ref[...], (((0,), (0,)), ((), ())),
                         preferred_element_type=jnp.float32)
    sum_ref[...] += jnp.sum(h0, axis=0, keepdims=True)
    sumsq_ref[...] += jnp.sum(h0 * h0, axis=0, keepdims=True)

    @pl.when(i == NSTEP - 1)
    def _():
        b1 = b1_ref[...]
        w2 = w2_ref[...]
        s0 = sum_ref[...]
        mean0 = s0 * (1.0 / B)
        mean = mean0 + b1
        var = sumsq_ref[...] * (1.0 / B) - mean0 * mean0
        c = gamma_ref[...] * w2 * lax.rsqrt(var + 1e-5)
        wvec_ref[...] = lax.dot_general(c, w1_ref[...], (((1,), (1,)), ((), ())),
                                        preferred_element_type=jnp.float32)
        k_ref[...] = b2_ref[...] + jnp.sum(
            beta_ref[...] * w2 + (b1 - mean) * c, axis=1, keepdims=True)


_stats_call = pl.pallas_call(
    _stats_body,
    grid=(NSTEP,),
    in_specs=[
        pl.BlockSpec((FD, TB), lambda i: (0, i)),
        pl.BlockSpec((FD, H), lambda i: (0, 0)),
        pl.BlockSpec((1, H), lambda i: (0, 0)),
        pl.BlockSpec((1, H), lambda i: (0, 0)),
        pl.BlockSpec((1, H), lambda i: (0, 0)),
        pl.BlockSpec((1, H), lambda i: (0, 0)),
        pl.BlockSpec((1, 1), lambda i: (0, 0)),
    ],
    out_specs=[
        pl.BlockSpec((1, FD), lambda i: (0, 0)),
        pl.BlockSpec((1, 1), lambda i: (0, 0)),
    ],
    out_shape=[
        jax.ShapeDtypeStruct((1, FD), jnp.float32),
        jax.ShapeDtypeStruct((1, 1), jnp.float32),
    ],
    scratch_shapes=[
        pltpu.VMEM((1, H), jnp.float32),
        pltpu.VMEM((1, H), jnp.float32),
    ],
    compiler_params=pltpu.CompilerParams(
        dimension_semantics=("arbitrary",)),
)


def _out_body(emb_ref, wvec_ref, k_ref, out_ref):
    z = lax.dot_general(wvec_ref[...], emb_ref[...], (((1,), (0,)), ((), ())),
                        preferred_element_type=jnp.float32)
    out_ref[...] = jax.nn.sigmoid(z + k_ref[0, 0])


_out_call = pl.pallas_call(
    _out_body,
    grid=(NSTEP,),
    in_specs=[
        pl.BlockSpec((FD, TB), lambda i: (0, i)),
        pl.BlockSpec((1, FD), lambda i: (0, 0)),
        pl.BlockSpec((1, 1), lambda i: (0, 0)),
    ],
    out_specs=pl.BlockSpec((1, TB), lambda i: (0, i)),
    out_shape=jax.ShapeDtypeStruct((1, B), jnp.float32),
)


def kernel(inputs, tables, W1, b1, gamma, beta, W2, b2):
    idx_t = inputs.astype(jnp.int32).T
    tab_t = jnp.transpose(tables, (0, 2, 1)).reshape(FD, V)
    emb_t = _get_sc_gather()(idx_t, tab_t)
    wvec, k = _stats_call(emb_t, W1, b1.reshape(1, H), gamma.reshape(1, H),
                          beta.reshape(1, H), W2.reshape(1, H),
                          b2.reshape(1, 1))
    out = _out_call(emb_t, wvec, k)
    return out.reshape(B, 1)


# speedup vs baseline: 11.3245x; 1.9913x over previous
import functools

import jax
import jax.numpy as jnp
import numpy as np
from jax import lax
from jax.experimental import pallas as pl
from jax.experimental.pallas import tpu as pltpu
from jax.experimental.pallas import tpu_sc as plsc

B = 16384
F = 26
V = 100000
D = 16
H = 300
FD = F * D

NC = 2
NS = 16
NW = NC * NS
PPW = FD // NW
QV = 4096
NQ = B // QV


VT = V // 128


def _sc_gather_body(idx_hbm, tab_hbm, out_hbm, idx_v, plane_v, stage_v,
                    sem0, sem1):
    w = lax.axis_index("s") * NC + lax.axis_index("c")
    p0 = w * PPW
    sems = (sem0, sem1)
    pending = [None, None]

    for j in range(PPW):
        p = p0 + j
        f = p // D
        d = p % D
        if j == 0:
            pltpu.sync_copy(idx_hbm.at[f], idx_v)
        else:
            f_prev = (p - 1) // D
            @pl.when(f != f_prev)
            def _():
                pltpu.sync_copy(idx_hbm.at[f], idx_v)
        pltpu.sync_copy(tab_hbm.at[f, d], plane_v)

        for q in range(NQ):
            s = (j * NQ + q) % 2
            if pending[s] is not None:
                pending[s].wait()

            def gq(i, carry):
                b = i * 32
                v0 = idx_v[pl.ds(q * QV + b, 16)]
                v1 = idx_v[pl.ds(q * QV + b + 16, 16)]
                stage_v[s, pl.ds(b, 16)] = plsc.load_gather(plane_v, [v0])
                stage_v[s, pl.ds(b + 16, 16)] = plsc.load_gather(plane_v, [v1])
                return carry

            lax.fori_loop(0, QV // 32, gq, 0)
            cp = pltpu.make_async_copy(
                stage_v.at[s], out_hbm.at[p, pl.ds(q * QV, QV)], sems[s])
            cp.start()
            pending[s] = cp

    for s in range(2):
        if pending[s] is not None:
            pending[s].wait()


@functools.cache
def _get_sc_gather():
    return pl.kernel(
        _sc_gather_body,
        out_type=jax.ShapeDtypeStruct((FD, B), jnp.float32),
        mesh=plsc.VectorSubcoreMesh(core_axis_name="c", subcore_axis_name="s"),
        scratch_types=[
            pltpu.VMEM((B,), jnp.int32),
            pltpu.VMEM((V,), jnp.float32),
            pltpu.VMEM((2, QV), jnp.float32),
            pltpu.SemaphoreType.DMA,
            pltpu.SemaphoreType.DMA,
        ],
        compiler_params=pltpu.CompilerParams(use_tc_tiling_on_sc=True,
                                             needs_layout_passes=False),
    )


TB = 2048
NSTEP = B // TB


def _stats_body(emb_ref, w1_ref, b1_ref, gamma_ref, beta_ref, w2_ref, b2_ref,
                wvec_ref, k_ref, sum_ref, sumsq_ref):
    i = pl.program_id(0)

    @pl.when(i == 0)
    def _():
        sum_ref[...] = jnp.zeros_like(sum_ref)
        sumsq_ref[...] = jnp.zeros_like(sumsq_ref)

    blk = emb_ref[...]
    h0 = lax.dot_general(blk, w1_ref[...], (((0,), (0,)), ((), ())),
                         preferred_element_type=jnp.float32)
    sum_ref[...] += jnp.sum(h0, axis=0, keepdims=True)
    sumsq_ref[...] += jnp.sum(h0 * h0, axis=0, keepdims=True)

    @pl.when(i == NSTEP - 1)
    def _():
        b1 = b1_ref[...]
        w2 = w2_ref[...]
        s0 = sum_ref[...]
        mean0 = s0 * (1.0 / B)
        mean = mean0 + b1
        var = sumsq_ref[...] * (1.0 / B) - mean0 * mean0
        c = gamma_ref[...] * w2 * lax.rsqrt(var + 1e-5)
        wvec_ref[...] = lax.dot_general(c, w1_ref[...], (((1,), (1,)), ((), ())),
                                        preferred_element_type=jnp.float32)
        k_ref[...] = b2_ref[...] + jnp.sum(
            beta_ref[...] * w2 + (b1 - mean) * c, axis=1, keepdims=True)


_stats_call = pl.pallas_call(
    _stats_body,
    grid=(NSTEP,),
    in_specs=[
        pl.BlockSpec((FD, TB), lambda i: (0, i)),
        pl.BlockSpec((FD, H), lambda i: (0, 0)),
        pl.BlockSpec((1, H), lambda i: (0, 0)),
        pl.BlockSpec((1, H), lambda i: (0, 0)),
        pl.BlockSpec((1, H), lambda i: (0, 0)),
        pl.BlockSpec((1, H), lambda i: (0, 0)),
        pl.BlockSpec((1, 1), lambda i: (0, 0)),
    ],
    out_specs=[
        pl.BlockSpec((1, FD), lambda i: (0, 0)),
        pl.BlockSpec((1, 1), lambda i: (0, 0)),
    ],
    out_shape=[
        jax.ShapeDtypeStruct((1, FD), jnp.float32),
        jax.ShapeDtypeStruct((1, 1), jnp.float32),
    ],
    scratch_shapes=[
        pltpu.VMEM((1, H), jnp.float32),
        pltpu.VMEM((1, H), jnp.float32),
    ],
    compiler_params=pltpu.CompilerParams(
        dimension_semantics=("arbitrary",)),
)


def _out_body(emb_ref, wvec_ref, k_ref, out_ref):
    z = lax.dot_general(wvec_ref[...], emb_ref[...], (((1,), (0,)), ((), ())),
                        preferred_element_type=jnp.float32)
    out_ref[...] = jax.nn.sigmoid(z + k_ref[0, 0])


_out_call = pl.pallas_call(
    _out_body,
    grid=(NSTEP,),
    in_specs=[
        pl.BlockSpec((FD, TB), lambda i: (0, i)),
        pl.BlockSpec((1, FD), lambda i: (0, 0)),
        pl.BlockSpec((1, 1), lambda i: (0, 0)),
    ],
    out_specs=pl.BlockSpec((1, TB), lambda i: (0, i)),
    out_shape=jax.ShapeDtypeStruct((1, B), jnp.float32),
)


def kernel(inputs, tables, W1, b1, gamma, beta, W2, b2):
    idx_t = inputs.astype(jnp.int32).T
    tab_t = jnp.transpose(tables, (0, 2, 1))
    emb_t = _get_sc_gather()(idx_t, tab_t)
    wvec, k = _stats_call(emb_t, W1, b1.reshape(1, H), gamma.reshape(1, H),
                          beta.reshape(1, H), W2.reshape(1, H),
                          b2.reshape(1, 1))
    out = _out_call(emb_t, wvec, k)
    return out.reshape(B, 1)


# speedup vs baseline: 16.4545x; 1.4530x over previous
import functools

import jax
import jax.numpy as jnp
import numpy as np
from jax import lax
from jax.experimental import pallas as pl
from jax.experimental.pallas import tpu as pltpu
from jax.experimental.pallas import tpu_sc as plsc

B = 16384
F = 26
V = 100000
D = 16
H = 300
FD = F * D

NC = 2
NS = 16
NW = NC * NS
PPW = FD // NW
QV = 4096
NQ = B // QV


VT = V // 128


def _sc_gather_body(idx_hbm, tab_hbm, out_hbm, idx_v, plane_v, stage_v,
                    sem0, sem1):
    w = lax.axis_index("s") * NC + lax.axis_index("c")
    p0 = w * PPW
    sems = (sem0, sem1)
    pending = [None, None]

    for j in range(PPW):
        p = p0 + j
        f = p // D
        d = p % D
        if j == 0:
            pltpu.sync_copy(idx_hbm.at[f], idx_v)
        else:
            f_prev = (p - 1) // D
            @pl.when(f != f_prev)
            def _():
                pltpu.sync_copy(idx_hbm.at[f], idx_v)
        pltpu.sync_copy(tab_hbm.at[f, d], plane_v)

        for q in range(NQ):
            s = (j * NQ + q) % 2
            if pending[s] is not None:
                pending[s].wait()

            @plsc.parallel_loop(0, QV, 32, unroll=8)
            def _(b):
                v0 = idx_v[pl.ds(q * QV + b, 16)]
                v1 = idx_v[pl.ds(q * QV + b + 16, 16)]
                stage_v[s, pl.ds(b, 16)] = plsc.load_gather(plane_v, [v0])
                stage_v[s, pl.ds(b + 16, 16)] = plsc.load_gather(plane_v, [v1])
            cp = pltpu.make_async_copy(
                stage_v.at[s], out_hbm.at[p, pl.ds(q * QV, QV)], sems[s])
            cp.start()
            pending[s] = cp

    for s in range(2):
        if pending[s] is not None:
            pending[s].wait()


@functools.cache
def _get_sc_gather():
    return pl.kernel(
        _sc_gather_body,
        out_type=jax.ShapeDtypeStruct((FD, B), jnp.float32),
        mesh=plsc.VectorSubcoreMesh(core_axis_name="c", subcore_axis_name="s"),
        scratch_types=[
            pltpu.VMEM((B,), jnp.int32),
            pltpu.VMEM((V,), jnp.float32),
            pltpu.VMEM((2, QV), jnp.float32),
            pltpu.SemaphoreType.DMA,
            pltpu.SemaphoreType.DMA,
        ],
        compiler_params=pltpu.CompilerParams(use_tc_tiling_on_sc=True,
                                             needs_layout_passes=False),
    )


TB = 2048
NSTEP = B // TB


def _stats_body(emb_ref, w1_ref, b1_ref, gamma_ref, beta_ref, w2_ref, b2_ref,
                wvec_ref, k_ref, sum_ref, sumsq_ref):
    i = pl.program_id(0)

    @pl.when(i == 0)
    def _():
        sum_ref[...] = jnp.zeros_like(sum_ref)
        sumsq_ref[...] = jnp.zeros_like(sumsq_ref)

    blk = emb_ref[...]
    h0 = lax.dot_general(blk, w1_ref[...], (((0,), (0,)), ((), ())),
                         preferred_element_type=jnp.float32)
    sum_ref[...] += jnp.sum(h0, axis=0, keepdims=True)
    sumsq_ref[...] += jnp.sum(h0 * h0, axis=0, keepdims=True)

    @pl.when(i == NSTEP - 1)
    def _():
        b1 = b1_ref[...]
        w2 = w2_ref[...]
        s0 = sum_ref[...]
        mean0 = s0 * (1.0 / B)
        mean = mean0 + b1
        var = sumsq_ref[...] * (1.0 / B) - mean0 * mean0
        c = gamma_ref[...] * w2 * lax.rsqrt(var + 1e-5)
        wvec_ref[...] = lax.dot_general(c, w1_ref[...], (((1,), (1,)), ((), ())),
                                        preferred_element_type=jnp.float32)
        k_ref[...] = b2_ref[...] + jnp.sum(
            beta_ref[...] * w2 + (b1 - mean) * c, axis=1, keepdims=True)


_stats_call = pl.pallas_call(
    _stats_body,
    grid=(NSTEP,),
    in_specs=[
        pl.BlockSpec((FD, TB), lambda i: (0, i)),
        pl.BlockSpec((FD, H), lambda i: (0, 0)),
        pl.BlockSpec((1, H), lambda i: (0, 0)),
        pl.BlockSpec((1, H), lambda i: (0, 0)),
        pl.BlockSpec((1, H), lambda i: (0, 0)),
        pl.BlockSpec((1, H), lambda i: (0, 0)),
        pl.BlockSpec((1, 1), lambda i: (0, 0)),
    ],
    out_specs=[
        pl.BlockSpec((1, FD), lambda i: (0, 0)),
        pl.BlockSpec((1, 1), lambda i: (0, 0)),
    ],
    out_shape=[
        jax.ShapeDtypeStruct((1, FD), jnp.float32),
        jax.ShapeDtypeStruct((1, 1), jnp.float32),
    ],
    scratch_shapes=[
        pltpu.VMEM((1, H), jnp.float32),
        pltpu.VMEM((1, H), jnp.float32),
    ],
    compiler_params=pltpu.CompilerParams(
        dimension_semantics=("arbitrary",)),
)


def _out_body(emb_ref, wvec_ref, k_ref, out_ref):
    z = lax.dot_general(wvec_ref[...], emb_ref[...], (((1,), (0,)), ((), ())),
                        preferred_element_type=jnp.float32)
    out_ref[...] = jax.nn.sigmoid(z + k_ref[0, 0])


_out_call = pl.pallas_call(
    _out_body,
    grid=(NSTEP,),
    in_specs=[
        pl.BlockSpec((FD, TB), lambda i: (0, i)),
        pl.BlockSpec((1, FD), lambda i: (0, 0)),
        pl.BlockSpec((1, 1), lambda i: (0, 0)),
    ],
    out_specs=pl.BlockSpec((1, TB), lambda i: (0, i)),
    out_shape=jax.ShapeDtypeStruct((1, B), jnp.float32),
)


def kernel(inputs, tables, W1, b1, gamma, beta, W2, b2):
    idx_t = inputs.astype(jnp.int32).T
    tab_t = jnp.transpose(tables, (0, 2, 1))
    emb_t = _get_sc_gather()(idx_t, tab_t)
    wvec, k = _stats_call(emb_t, W1, b1.reshape(1, H), gamma.reshape(1, H),
                          beta.reshape(1, H), W2.reshape(1, H),
                          b2.reshape(1, 1))
    out = _out_call(emb_t, wvec, k)
    return out.reshape(B, 1)
